# baseline (speedup 1.0000x reference)
import functools

import jax
import jax.numpy as jnp
from jax import lax
from jax.experimental import pallas as pl
from jax.experimental.pallas import tpu as pltpu
from jax.experimental.pallas import tpu_sc as plsc

N = 10000
E = 160000
SI, VI = 128, 16
SE, VE = 16, 1
SO, VO = 128, 16
H0 = 2 * VI + VE
H0P = 48
ROW = 192

EDGE_BLK = 2000
NODE_BLK = 1000


def _pad2(a, r, c):
    return jnp.pad(a, ((0, r - a.shape[0]), (0, c - a.shape[1])))


TBL = 176


def _node_kernel(xs_ref, xv_ref, wsrc_ref, wdst_ref, osrc_ref, odst_ref):
    xs = xs_ref[...]
    xv = xv_ref[...]
    osrc_ref[:, 0:128] = jnp.dot(xs, wsrc_ref[...], preferred_element_type=jnp.float32)
    osrc_ref[:, 128:176] = xv
    odst_ref[:, 0:128] = jnp.dot(xs, wdst_ref[...], preferred_element_type=jnp.float32)
    odst_ref[:, 128:176] = xv


def _node_tables(x_s, xv48, w_ssrc, w_sdst, interpret=False):
    grid = (N // NODE_BLK,)
    return pl.pallas_call(
        _node_kernel,
        grid=grid,
        in_specs=[
            pl.BlockSpec((NODE_BLK, SI), lambda i: (i, 0)),
            pl.BlockSpec((NODE_BLK, 48), lambda i: (i, 0)),
            pl.BlockSpec((SI, SO), lambda i: (0, 0)),
            pl.BlockSpec((SI, SO), lambda i: (0, 0)),
        ],
        out_specs=[
            pl.BlockSpec((NODE_BLK, TBL), lambda i: (i, 0)),
            pl.BlockSpec((NODE_BLK, TBL), lambda i: (i, 0)),
        ],
        out_shape=[
            jax.ShapeDtypeStruct((N, TBL), jnp.float32),
            jax.ShapeDtypeStruct((N, TBL), jnp.float32),
        ],
        interpret=interpret,
    )(x_s, xv48, w_ssrc, w_sdst)


GK = 128
NCHUNK = E // GK
_NC, _NS = 2, 16
_NW = _NC * _NS
_ITERS = (NCHUNK + _NW - 1) // _NW


_BASE_CH = NCHUNK // _NW
_EXTRA = NCHUNK - _BASE_CH * _NW


def _gather_body(nchunk, tsrc, tdst, src_hbm, dst_hbm, out_s, out_v,
                 idx_s0, idx_d0, idx_s1, idx_d1, bs0, bd0, bs1, bd1,
                 gsem0, gsem1, osem0, osem1):
    base = nchunk // _NW
    extra = nchunk % _NW
    wid = lax.axis_index("s") * _NC + lax.axis_index("c")
    nc = jnp.where(wid < extra, base + 1, base)
    start = base * wid + jnp.minimum(wid, extra)

    idx_s = (idx_s0, idx_s1)
    idx_d = (idx_d0, idx_d1)
    bs = (bs0, bs1)
    bd = (bd0, bd1)
    gsem = (gsem0, gsem1)
    osem = (osem0, osem1)

    def load_idx(c, p):
        off = (start + c) * GK
        pltpu.sync_copy(src_hbm.at[pl.ds(off, GK)], idx_s[p])
        pltpu.sync_copy(dst_hbm.at[pl.ds(off, GK)], idx_d[p])

    def start_gather(p):
        pltpu.async_copy(tsrc.at[idx_s[p]], bs[p], gsem[p])
        pltpu.async_copy(tdst.at[idx_d[p]], bd[p], gsem[p])

    def wait_gather(p):
        pltpu.make_async_copy(tsrc.at[idx_s[p]], bs[p], gsem[p]).wait()
        pltpu.make_async_copy(tdst.at[idx_d[p]], bd[p], gsem[p]).wait()

    def tec(p):
        b_s, b_d = bs[p], bd[p]

        zero16 = jnp.zeros((16,), jnp.float32)

        def row(k, _):
            for l in range(8):
                sl = pl.ds(16 * l, 16)
                b_s[k, sl] = b_s[k, sl] + b_d[k, sl]
            for l in range(3):
                s_sl = pl.ds(128 + 16 * l, 16)
                b_d[k, pl.ds(16 * l, 16)] = b_s[k, s_sl]
                b_d[k, pl.ds(48 + 16 * l, 16)] = b_d[k, s_sl]
            b_d[k, pl.ds(96, 16)] = zero16
            b_d[k, pl.ds(112, 16)] = zero16
            return 0

        lax.fori_loop(0, GK, row, 0)

    def start_out(c, p):
        off = (start + c) * GK
        pltpu.async_copy(bs[p].at[:, pl.ds(0, 128)],
                         out_s.at[pl.ds(off, GK)], osem[p])
        pltpu.async_copy(bd[p].at[:, pl.ds(0, 128)],
                         out_v.at[pl.ds(off, GK)], osem[p])

    def wait_out(p):
        pltpu.make_async_copy(bs[p].at[:, pl.ds(0, 128)],
                              out_s.at[pl.ds(0, GK)], osem[p]).wait()
        pltpu.make_async_copy(bd[p].at[:, pl.ds(0, 128)],
                              out_v.at[pl.ds(0, GK)], osem[p]).wait()

    load_idx(0, 0)
    start_gather(0)

    def half(i, p):
        @pl.when(i < nc)
        def _():
            @pl.when(i >= 1)
            def _():
                wait_out(1 - p)

            @pl.when(i + 1 < nc)
            def _():
                load_idx(i + 1, 1 - p)
                start_gather(1 - p)

            wait_gather(p)
            tec(p)
            start_out(i, p)

    def body2(i2, _):
        half(2 * i2, 0)
        half(2 * i2 + 1, 1)
        return 0

    maxnc = base + (1 if extra else 0)
    lax.fori_loop(0, (maxnc + 1) // 2, body2, 0)

    last = (nc - 1) % 2

    @pl.when(last == 0)
    def _():
        wait_out(0)

    @pl.when(last == 1)
    def _():
        wait_out(1)


def _sc_gather(tsrc, tdst, src, dst, e):
    f32 = jnp.float32
    return pl.kernel(
        functools.partial(_gather_body, e // GK),
        out_type=[
            jax.ShapeDtypeStruct((e, 128), f32),
            jax.ShapeDtypeStruct((e, 128), f32),
        ],
        mesh=plsc.VectorSubcoreMesh(core_axis_name="c", subcore_axis_name="s"),
        scratch_types=[
            pltpu.VMEM((GK,), jnp.int32),
            pltpu.VMEM((GK,), jnp.int32),
            pltpu.VMEM((GK,), jnp.int32),
            pltpu.VMEM((GK,), jnp.int32),
            pltpu.VMEM((GK, TBL), f32),
            pltpu.VMEM((GK, TBL), f32),
            pltpu.VMEM((GK, TBL), f32),
            pltpu.VMEM((GK, TBL), f32),
            pltpu.SemaphoreType.DMA,
            pltpu.SemaphoreType.DMA,
            pltpu.SemaphoreType.DMA,
            pltpu.SemaphoreType.DMA,
        ],
        compiler_params=pltpu.CompilerParams(use_tc_tiling_on_sc=False),
    )(tsrc, tdst, src, dst)


def _edge_kernel(gs_ref, gv_ref, eas_ref, eav_ref,
                 w_se_ref, b0_ref, wbig_ref, we_ref, s0m_ref, wsvn_ref,
                 wv0_ref, wsv0_ref, bsv0_ref,
                 wh1_ref, ws1_ref, wvn1_ref, b1_ref, wv1_ref, wsv1_ref, bsv1_ref,
                 s1m_ref,
                 wh2_ref, ws2_ref, wvn2_ref, b2_ref, wv2_ref, wsv2_ref, bsv2_ref,
                 out1_ref, out2_ref):
    f32 = jnp.float32

    def dot(a, b):
        return jnp.dot(a, b, preferred_element_type=f32)

    def sig(x):
        return 1.0 / (1.0 + jnp.exp(-x))

    gs = gs_ref[...]
    gv = gv_ref[...]
    eas = eas_ref[...]
    eav = eav_ref[...]

    vh = dot(gv, wbig_ref[...]) + dot(eav, we_ref[...])
    vn = jnp.sqrt(jnp.clip(dot(vh * vh, s0m_ref[...]), 1e-8, None))
    s0 = (gs + dot(eas, w_se_ref[...]) + dot(vn, wsvn_ref[...]) + b0_ref[...])
    gate0 = sig(dot(sig(s0), wsv0_ref[...]) + bsv0_ref[...])
    v0 = dot(vh, wv0_ref[...]) * gate0
    s0 = jax.nn.relu(s0)

    vh1 = dot(v0, wh1_ref[...])
    vn1 = jnp.sqrt(jnp.clip(dot(vh1 * vh1, s1m_ref[...]), 1e-8, None))
    s1 = dot(s0, ws1_ref[...]) + dot(vn1, wvn1_ref[...]) + b1_ref[...]
    gate1 = sig(dot(sig(s1), wsv1_ref[...]) + bsv1_ref[...])
    v1 = dot(vh1, wv1_ref[...]) * gate1
    s1 = jax.nn.relu(s1)

    vh2 = dot(v1, wh2_ref[...])
    vn2 = jnp.sqrt(jnp.clip(dot(vh2 * vh2, s1m_ref[...]), 1e-8, None))
    s2 = dot(s1, ws2_ref[...]) + dot(vn2, wvn2_ref[...]) + b2_ref[...]
    gate2 = sig(dot(s2, wsv2_ref[...]) + bsv2_ref[...])
    v2 = dot(vh2, wv2_ref[...]) * gate2

    out1_ref[...] = s2
    out2_ref[...] = jnp.concatenate(
        [v2, jnp.ones((s2.shape[0], 16), f32),
         jnp.zeros((s2.shape[0], 64), f32)], axis=1)


def _edge_gvp(gs, gv, eas, eav, wts, e=E, interpret=False):
    B = EDGE_BLK
    grid = (e // B,)
    full = lambda s: pl.BlockSpec(s, lambda i: (0, 0))
    in_specs = [
        pl.BlockSpec((B, SO), lambda i: (i, 0)),
        pl.BlockSpec((B, 128), lambda i: (i, 0)),
        pl.BlockSpec((B, SE), lambda i: (i, 0)),
        pl.BlockSpec((B, 3), lambda i: (i, 0)),
    ] + [full(w.shape) for w in wts]
    return pl.pallas_call(
        _edge_kernel,
        grid=grid,
        in_specs=in_specs,
        out_specs=[
            pl.BlockSpec((B, 128), lambda i: (i, 0)),
            pl.BlockSpec((B, 128), lambda i: (i, 0)),
        ],
        out_shape=[
            jax.ShapeDtypeStruct((e, 128), jnp.float32),
            jax.ShapeDtypeStruct((e, 128), jnp.float32),
        ],
        interpret=interpret,
    )(gs, gv, eas, eav, *wts)


SK = 128
SCHUNKS = E // SK
_SITERS = (SCHUNKS + _NS - 1) // _NS
NPT = N // _NS
NZC = 125


def _scatter_body(nhchunk, m1a, m1b, m2a, m2b, d2a, d2b, o1, o2,
                  idx0, idx1, buf0, buf1, stage, acc, msem0, msem1):
    sbase = nhchunk // _NS
    sextra = nhchunk % _NS
    c = lax.axis_index("c")
    s = lax.axis_index("s")
    nc = jnp.where(s < sextra, sbase + 1, sbase)
    start = sbase * s + jnp.minimum(s, sextra)

    idx = (idx0, idx1)
    buf = (buf0, buf1)
    msem = (msem0, msem1)

    def zrow(k, _):
        for l in range(8):
            stage[k, pl.ds(16 * l, 16)] = jnp.zeros((16,), jnp.float32)
        return 0

    lax.fori_loop(0, NZC, zrow, 0)
    for j in range(NPT // NZC):
        pltpu.sync_copy(stage, acc.at[pl.ds(NPT * s + NZC * j, NZC)])
    plsc.subcore_barrier()

    def accumulate(m, d2):
        def load(i, p):
            cid = start + i
            pltpu.sync_copy(d2.at[cid], idx[p])
            pltpu.async_copy(m.at[pl.ds(cid * SK, SK)], buf[p], msem[p])

        def waitm(p):
            pltpu.make_async_copy(m.at[pl.ds(0, SK)], buf[p], msem[p]).wait()

        load(0, 0)

        def half(i, p):
            @pl.when(i < nc)
            def _():
                @pl.when(i + 1 < nc)
                def _():
                    load(i + 1, 1 - p)

                waitm(p)
                pltpu.sync_copy(buf[p], acc.at[idx[p]], add=True)

        def body2(i2, _):
            half(2 * i2, 0)
            half(2 * i2 + 1, 1)
            return 0

        maxnc = sbase + (1 if sextra else 0)
        lax.fori_loop(0, (maxnc + 1) // 2, body2, 0)

    @pl.when(c == 0)
    def _():
        accumulate(m1a, d2a)
        accumulate(m1b, d2b)

    @pl.when(c == 1)
    def _():
        accumulate(m2a, d2a)
        accumulate(m2b, d2b)

    plsc.subcore_barrier()

    def writeout(o):
        for j in range(NPT // NZC):
            sl = pl.ds(NPT * s + NZC * j, NZC)
            pltpu.sync_copy(acc.at[sl], stage)
            pltpu.sync_copy(stage, o.at[sl])

    @pl.when(c == 0)
    def _():
        writeout(o1)

    @pl.when(c == 1)
    def _():
        writeout(o2)


def _sc_scatter(m1a, m1b, m2a, m2b, d2a, d2b, nhchunk):
    f32 = jnp.float32
    return pl.kernel(
        functools.partial(_scatter_body, nhchunk),
        out_type=[
            jax.ShapeDtypeStruct((N, 128), f32),
            jax.ShapeDtypeStruct((N, 128), f32),
        ],
        mesh=plsc.VectorSubcoreMesh(core_axis_name="c", subcore_axis_name="s"),
        scratch_types=[
            pltpu.VMEM((SK,), jnp.int32),
            pltpu.VMEM((SK,), jnp.int32),
            pltpu.VMEM((SK, 128), f32),
            pltpu.VMEM((SK, 128), f32),
            pltpu.VMEM((NZC, 128), f32),
            pltpu.VMEM_SHARED((N, 128), f32),
            pltpu.SemaphoreType.DMA,
            pltpu.SemaphoreType.DMA,
        ],
        compiler_params=pltpu.CompilerParams(use_tc_tiling_on_sc=False),
    )(m1a, m1b, m2a, m2b, d2a, d2b)


def _combine_kernel(p1_ref, p2_ref, xs_ref, xv_ref, os_ref, ov_ref):
    p1 = p1_ref[...]
    p2 = p2_ref[...]
    cnt = jnp.clip(p2[:, 48:49], 1.0, None)
    recip = 1.0 / cnt
    os_ref[...] = xs_ref[...] + p1 * recip
    ov_ref[...] = xv_ref[...] + p2[:, 0:48] * recip


def _combine(p1, p2, x_s, xv48, interpret=False):
    grid = (N // NODE_BLK,)
    return pl.pallas_call(
        _combine_kernel,
        grid=grid,
        in_specs=[
            pl.BlockSpec((NODE_BLK, 128), lambda i: (i, 0)),
            pl.BlockSpec((NODE_BLK, 128), lambda i: (i, 0)),
            pl.BlockSpec((NODE_BLK, SI), lambda i: (i, 0)),
            pl.BlockSpec((NODE_BLK, 48), lambda i: (i, 0)),
        ],
        out_specs=[
            pl.BlockSpec((NODE_BLK, SI), lambda i: (i, 0)),
            pl.BlockSpec((NODE_BLK, 48), lambda i: (i, 0)),
        ],
        out_shape=[
            jax.ShapeDtypeStruct((N, SI), jnp.float32),
            jax.ShapeDtypeStruct((N, 48), jnp.float32),
        ],
        interpret=interpret,
    )(p1, p2, x_s, xv48)


def _split_weights(params):
    p0, p1, p2 = params['layer0'], params['layer1'], params['layer2']
    eye3 = jnp.eye(3, dtype=jnp.float32)
    ws0 = p0['ws_w']
    w_ssrc = ws0[:SI]
    w_se = ws0[SI:SI + SE]
    w_sdst = ws0[SI + SE:SI + SE + SI]
    w_svn = _pad2(ws0[SI + SE + SI:], H0P, SO)
    wh0 = p0['wh']
    whs = _pad2(wh0[:VI], VI, H0P)
    whe = _pad2(wh0[VI:VI + VE], VE, H0P)
    whd = _pad2(wh0[VI + VE:], VI, H0P)
    wv0 = _pad2(p0['wv'], H0P, VO)
    wbig = jnp.concatenate([jnp.kron(eye3, whs), jnp.kron(eye3, whd),
                            jnp.zeros((32, 3 * H0P), jnp.float32)], axis=0)
    we = jnp.kron(eye3, whe)
    s0m = jnp.tile(jnp.eye(H0P, dtype=jnp.float32), (3, 1))
    s1m = jnp.tile(jnp.eye(VO, dtype=jnp.float32), (3, 1))
    wts = (
        w_se, p0['ws_b'][None, :], wbig, we, s0m, w_svn,
        jnp.kron(eye3, wv0),
        jnp.tile(p0['wsv_w'], (1, 3)),
        jnp.tile(p0['wsv_b'], (3,))[None, :],
        jnp.kron(eye3, p1['wh']),
        p1['ws_w'][:SO], p1['ws_w'][SO:], p1['ws_b'][None, :],
        jnp.kron(eye3, p1['wv']),
        jnp.tile(p1['wsv_w'], (1, 3)), jnp.tile(p1['wsv_b'], (3,))[None, :],
        s1m,
        jnp.kron(eye3, p2['wh']),
        p2['ws_w'][:SO], p2['ws_w'][SO:], p2['ws_b'][None, :],
        jnp.kron(eye3, p2['wv']),
        jnp.tile(p2['wsv_w'], (1, 3)), jnp.tile(p2['wsv_b'], (3,))[None, :],
    )
    return w_ssrc, w_sdst, wts


def kernel(x_s, x_v, edge_index, edge_attr_s, edge_attr_v, params):
    src, dst = edge_index[0], edge_index[1]
    w_ssrc, w_sdst, wts = _split_weights(params)

    xv48 = jnp.swapaxes(x_v, 1, 2).reshape(N, 48)
    ts_src, ts_dst = _node_tables(x_s, xv48, w_ssrc, w_sdst)
    eav = edge_attr_v.reshape(E, 3)

    EH = E // 2
    ms = []
    for k in range(2):
        sl = slice(k * EH, (k + 1) * EH)
        gs, gv = _sc_gather(ts_src, ts_dst, src[sl], dst[sl], EH)
        m1, m2 = _edge_gvp(gs, gv, edge_attr_s[sl], eav[sl], wts, EH)
        ms.append((m1, m2))

    p1, p2 = _sc_scatter(ms[0][0], ms[1][0], ms[0][1], ms[1][1],
                         dst[:EH].reshape(EH // SK, SK),
                         dst[EH:].reshape(EH // SK, SK), EH // SK)

    out_s, out_v48 = _combine(p1, p2, x_s, xv48)
    out_v = jnp.swapaxes(out_v48.reshape(N, 3, VI), 1, 2)
    return (out_s, out_v)


# speedup vs baseline: 1.0886x; 1.0886x over previous
import functools

import jax
import jax.numpy as jnp
from jax import lax
from jax.experimental import pallas as pl
from jax.experimental.pallas import tpu as pltpu
from jax.experimental.pallas import tpu_sc as plsc

N = 10000
E = 160000
SI, VI = 128, 16
SE, VE = 16, 1
SO, VO = 128, 16
H0 = 2 * VI + VE
H0P = 48
ROW = 192

EDGE_BLK = 2000
NODE_BLK = 1000


def _pad2(a, r, c):
    return jnp.pad(a, ((0, r - a.shape[0]), (0, c - a.shape[1])))


TBL = 176


def _node_kernel(xs_ref, xv_ref, wsrc_ref, wdst_ref, osrc_ref, odst_ref):
    xs = xs_ref[...]
    xv = xv_ref[...]
    osrc_ref[:, 0:128] = jnp.dot(xs, wsrc_ref[...], preferred_element_type=jnp.float32)
    osrc_ref[:, 128:176] = xv
    odst_ref[:, 0:128] = jnp.dot(xs, wdst_ref[...], preferred_element_type=jnp.float32)
    odst_ref[:, 128:176] = xv


def _node_tables(x_s, xv48, w_ssrc, w_sdst, interpret=False):
    grid = (N // NODE_BLK,)
    return pl.pallas_call(
        _node_kernel,
        grid=grid,
        in_specs=[
            pl.BlockSpec((NODE_BLK, SI), lambda i: (i, 0)),
            pl.BlockSpec((NODE_BLK, 48), lambda i: (i, 0)),
            pl.BlockSpec((SI, SO), lambda i: (0, 0)),
            pl.BlockSpec((SI, SO), lambda i: (0, 0)),
        ],
        out_specs=[
            pl.BlockSpec((NODE_BLK, TBL), lambda i: (i, 0)),
            pl.BlockSpec((NODE_BLK, TBL), lambda i: (i, 0)),
        ],
        out_shape=[
            jax.ShapeDtypeStruct((N, TBL), jnp.float32),
            jax.ShapeDtypeStruct((N, TBL), jnp.float32),
        ],
        interpret=interpret,
    )(x_s, xv48, w_ssrc, w_sdst)


GK = 128
NCHUNK = E // GK
_NC, _NS = 2, 16
_NW = _NC * _NS
_ITERS = (NCHUNK + _NW - 1) // _NW


_BASE_CH = NCHUNK // _NW
_EXTRA = NCHUNK - _BASE_CH * _NW


def _gather_body(nchunk, tsrc, tdst, src_hbm, dst_hbm, out_s, out_v,
                 idx_s0, idx_d0, idx_s1, idx_d1, bs0, bd0, bs1, bd1,
                 gsem0, gsem1, osem0, osem1):
    base = nchunk // _NW
    extra = nchunk % _NW
    wid = lax.axis_index("s") * _NC + lax.axis_index("c")
    nc = jnp.where(wid < extra, base + 1, base)
    start = base * wid + jnp.minimum(wid, extra)

    idx_s = (idx_s0, idx_s1)
    idx_d = (idx_d0, idx_d1)
    bs = (bs0, bs1)
    bd = (bd0, bd1)
    gsem = (gsem0, gsem1)
    osem = (osem0, osem1)

    def load_idx(c, p):
        off = (start + c) * GK
        pltpu.sync_copy(src_hbm.at[pl.ds(off, GK)], idx_s[p])
        pltpu.sync_copy(dst_hbm.at[pl.ds(off, GK)], idx_d[p])

    def start_gather(p):
        pltpu.async_copy(tsrc.at[idx_s[p]], bs[p], gsem[p])
        pltpu.async_copy(tdst.at[idx_d[p]], bd[p], gsem[p])

    def wait_gather(p):
        pltpu.make_async_copy(tsrc.at[idx_s[p]], bs[p], gsem[p]).wait()
        pltpu.make_async_copy(tdst.at[idx_d[p]], bd[p], gsem[p]).wait()

    def tec(p):
        b_s, b_d = bs[p], bd[p]

        zero16 = jnp.zeros((16,), jnp.float32)

        def row(k, _):
            for l in range(8):
                sl = pl.ds(16 * l, 16)
                b_s[k, sl] = b_s[k, sl] + b_d[k, sl]
            for l in range(3):
                s_sl = pl.ds(128 + 16 * l, 16)
                b_d[k, pl.ds(16 * l, 16)] = b_s[k, s_sl]
                b_d[k, pl.ds(48 + 16 * l, 16)] = b_d[k, s_sl]
            b_d[k, pl.ds(96, 16)] = zero16
            b_d[k, pl.ds(112, 16)] = zero16
            return 0

        lax.fori_loop(0, GK, row, 0)

    def start_out(c, p):
        off = (start + c) * GK
        pltpu.async_copy(bs[p].at[:, pl.ds(0, 128)],
                         out_s.at[pl.ds(off, GK)], osem[p])
        pltpu.async_copy(bd[p].at[:, pl.ds(0, 128)],
                         out_v.at[pl.ds(off, GK)], osem[p])

    def wait_out(p):
        pltpu.make_async_copy(bs[p].at[:, pl.ds(0, 128)],
                              out_s.at[pl.ds(0, GK)], osem[p]).wait()
        pltpu.make_async_copy(bd[p].at[:, pl.ds(0, 128)],
                              out_v.at[pl.ds(0, GK)], osem[p]).wait()

    load_idx(0, 0)
    start_gather(0)

    def half(i, p):
        @pl.when(i < nc)
        def _():
            @pl.when(i >= 1)
            def _():
                wait_out(1 - p)

            @pl.when(i + 1 < nc)
            def _():
                load_idx(i + 1, 1 - p)
                start_gather(1 - p)

            wait_gather(p)
            tec(p)
            start_out(i, p)

    def body2(i2, _):
        half(2 * i2, 0)
        half(2 * i2 + 1, 1)
        return 0

    maxnc = base + (1 if extra else 0)
    lax.fori_loop(0, (maxnc + 1) // 2, body2, 0)

    last = (nc - 1) % 2

    @pl.when(last == 0)
    def _():
        wait_out(0)

    @pl.when(last == 1)
    def _():
        wait_out(1)


def _sc_gather(tsrc, tdst, src, dst, e):
    f32 = jnp.float32
    return pl.kernel(
        functools.partial(_gather_body, e // GK),
        out_type=[
            jax.ShapeDtypeStruct((e, 128), f32),
            jax.ShapeDtypeStruct((e, 128), f32),
        ],
        mesh=plsc.VectorSubcoreMesh(core_axis_name="c", subcore_axis_name="s"),
        scratch_types=[
            pltpu.VMEM((GK,), jnp.int32),
            pltpu.VMEM((GK,), jnp.int32),
            pltpu.VMEM((GK,), jnp.int32),
            pltpu.VMEM((GK,), jnp.int32),
            pltpu.VMEM((GK, TBL), f32),
            pltpu.VMEM((GK, TBL), f32),
            pltpu.VMEM((GK, TBL), f32),
            pltpu.VMEM((GK, TBL), f32),
            pltpu.SemaphoreType.DMA,
            pltpu.SemaphoreType.DMA,
            pltpu.SemaphoreType.DMA,
            pltpu.SemaphoreType.DMA,
        ],
        compiler_params=pltpu.CompilerParams(use_tc_tiling_on_sc=False),
    )(tsrc, tdst, src, dst)


def _edge_kernel(gs_ref, gv_ref, eas_ref, eav_ref,
                 w_se_ref, b0_ref, wbig_ref, we_ref, s0m_ref, wsvn_ref,
                 wv0_ref, wsv0_ref, bsv0_ref,
                 wh1_ref, ws1_ref, wvn1_ref, b1_ref, wv1_ref, wsv1_ref, bsv1_ref,
                 s1m_ref,
                 wh2_ref, ws2_ref, wvn2_ref, b2_ref, wv2_ref, wsv2_ref, bsv2_ref,
                 out1_ref, out2_ref):
    f32 = jnp.float32

    def dot(a, b):
        return jnp.dot(a, b, preferred_element_type=f32)

    def sig(x):
        return 1.0 / (1.0 + jnp.exp(-x))

    gs = gs_ref[...]
    gv = gv_ref[...]
    eas = eas_ref[...]
    eav = eav_ref[...]

    vh = dot(gv, wbig_ref[...]) + dot(eav, we_ref[...])
    vn = jnp.sqrt(jnp.clip(dot(vh * vh, s0m_ref[...]), 1e-8, None))
    s0 = (gs + dot(eas, w_se_ref[...]) + dot(vn, wsvn_ref[...]) + b0_ref[...])
    gate0 = sig(dot(sig(s0), wsv0_ref[...]) + bsv0_ref[...])
    v0 = dot(vh, wv0_ref[...]) * gate0
    s0 = jax.nn.relu(s0)

    vh1 = dot(v0, wh1_ref[...])
    vn1 = jnp.sqrt(jnp.clip(dot(vh1 * vh1, s1m_ref[...]), 1e-8, None))
    s1 = dot(s0, ws1_ref[...]) + dot(vn1, wvn1_ref[...]) + b1_ref[...]
    gate1 = sig(dot(sig(s1), wsv1_ref[...]) + bsv1_ref[...])
    v1 = dot(vh1, wv1_ref[...]) * gate1
    s1 = jax.nn.relu(s1)

    vh2 = dot(v1, wh2_ref[...])
    vn2 = jnp.sqrt(jnp.clip(dot(vh2 * vh2, s1m_ref[...]), 1e-8, None))
    s2 = dot(s1, ws2_ref[...]) + dot(vn2, wvn2_ref[...]) + b2_ref[...]
    gate2 = sig(dot(s2, wsv2_ref[...]) + bsv2_ref[...])
    v2 = dot(vh2, wv2_ref[...]) * gate2

    out1_ref[...] = s2
    out2_ref[...] = jnp.concatenate(
        [v2, jnp.ones((s2.shape[0], 16), f32),
         jnp.zeros((s2.shape[0], 64), f32)], axis=1)


def _edge_gvp(gs, gv, eas, eav, wts, e=E, interpret=False):
    B = EDGE_BLK
    grid = (e // B,)
    full = lambda s: pl.BlockSpec(s, lambda i: (0, 0))
    in_specs = [
        pl.BlockSpec((B, SO), lambda i: (i, 0)),
        pl.BlockSpec((B, 128), lambda i: (i, 0)),
        pl.BlockSpec((B, SE), lambda i: (i, 0)),
        pl.BlockSpec((B, 3), lambda i: (i, 0)),
    ] + [full(w.shape) for w in wts]
    return pl.pallas_call(
        _edge_kernel,
        grid=grid,
        in_specs=in_specs,
        out_specs=[
            pl.BlockSpec((B, 128), lambda i: (i, 0)),
            pl.BlockSpec((B, 128), lambda i: (i, 0)),
        ],
        out_shape=[
            jax.ShapeDtypeStruct((e, 128), jnp.float32),
            jax.ShapeDtypeStruct((e, 128), jnp.float32),
        ],
        interpret=interpret,
    )(gs, gv, eas, eav, *wts)


SK = 128
SCHUNKS = E // SK
_SITERS = (SCHUNKS + _NS - 1) // _NS
NPT = N // _NS
NZC = 125


def _scatter_body(nhchunk, m1, m2, d2, o1, o2,
                  idx0, idx1, buf0, buf1, stage, acc, msem0, msem1):
    sbase = nhchunk // _NS
    sextra = nhchunk % _NS
    c = lax.axis_index("c")
    s = lax.axis_index("s")
    nc = jnp.where(s < sextra, sbase + 1, sbase)
    start = sbase * s + jnp.minimum(s, sextra)

    idx = (idx0, idx1)
    buf = (buf0, buf1)
    msem = (msem0, msem1)

    def zrow(k, _):
        for l in range(8):
            stage[k, pl.ds(16 * l, 16)] = jnp.zeros((16,), jnp.float32)
        return 0

    lax.fori_loop(0, NZC, zrow, 0)
    for j in range(NPT // NZC):
        pltpu.sync_copy(stage, acc.at[pl.ds(NPT * s + NZC * j, NZC)])
    plsc.subcore_barrier()

    def accumulate(m):
        def load(i, p):
            cid = start + i
            pltpu.sync_copy(d2.at[cid], idx[p])
            pltpu.async_copy(m.at[pl.ds(cid * SK, SK)], buf[p], msem[p])

        def waitm(p):
            pltpu.make_async_copy(m.at[pl.ds(0, SK)], buf[p], msem[p]).wait()

        load(0, 0)

        def half(i, p):
            @pl.when(i < nc)
            def _():
                @pl.when(i + 1 < nc)
                def _():
                    load(i + 1, 1 - p)

                waitm(p)
                pltpu.sync_copy(buf[p], acc.at[idx[p]], add=True)

        def body2(i2, _):
            half(2 * i2, 0)
            half(2 * i2 + 1, 1)
            return 0

        maxnc = sbase + (1 if sextra else 0)
        lax.fori_loop(0, (maxnc + 1) // 2, body2, 0)

    @pl.when(c == 0)
    def _():
        accumulate(m1)

    @pl.when(c == 1)
    def _():
        accumulate(m2)

    plsc.subcore_barrier()

    def writeout(o):
        for j in range(NPT // NZC):
            sl = pl.ds(NPT * s + NZC * j, NZC)
            pltpu.sync_copy(acc.at[sl], stage)
            pltpu.sync_copy(stage, o.at[sl])

    @pl.when(c == 0)
    def _():
        writeout(o1)

    @pl.when(c == 1)
    def _():
        writeout(o2)


def _sc_scatter(m1, m2, d2):
    f32 = jnp.float32
    return pl.kernel(
        functools.partial(_scatter_body, SCHUNKS),
        out_type=[
            jax.ShapeDtypeStruct((N, 128), f32),
            jax.ShapeDtypeStruct((N, 128), f32),
        ],
        mesh=plsc.VectorSubcoreMesh(core_axis_name="c", subcore_axis_name="s"),
        scratch_types=[
            pltpu.VMEM((SK,), jnp.int32),
            pltpu.VMEM((SK,), jnp.int32),
            pltpu.VMEM((SK, 128), f32),
            pltpu.VMEM((SK, 128), f32),
            pltpu.VMEM((NZC, 128), f32),
            pltpu.VMEM_SHARED((N, 128), f32),
            pltpu.SemaphoreType.DMA,
            pltpu.SemaphoreType.DMA,
        ],
        compiler_params=pltpu.CompilerParams(use_tc_tiling_on_sc=False),
    )(m1, m2, d2)


def _combine_kernel(p1_ref, p2_ref, xs_ref, xv_ref, os_ref, ov_ref):
    p1 = p1_ref[...]
    p2 = p2_ref[...]
    cnt = jnp.clip(p2[:, 48:49], 1.0, None)
    recip = 1.0 / cnt
    os_ref[...] = xs_ref[...] + p1 * recip
    ov_ref[...] = xv_ref[...] + p2[:, 0:48] * recip


def _combine(p1, p2, x_s, xv48, interpret=False):
    grid = (N // NODE_BLK,)
    return pl.pallas_call(
        _combine_kernel,
        grid=grid,
        in_specs=[
            pl.BlockSpec((NODE_BLK, 128), lambda i: (i, 0)),
            pl.BlockSpec((NODE_BLK, 128), lambda i: (i, 0)),
            pl.BlockSpec((NODE_BLK, SI), lambda i: (i, 0)),
            pl.BlockSpec((NODE_BLK, 48), lambda i: (i, 0)),
        ],
        out_specs=[
            pl.BlockSpec((NODE_BLK, SI), lambda i: (i, 0)),
            pl.BlockSpec((NODE_BLK, 48), lambda i: (i, 0)),
        ],
        out_shape=[
            jax.ShapeDtypeStruct((N, SI), jnp.float32),
            jax.ShapeDtypeStruct((N, 48), jnp.float32),
        ],
        interpret=interpret,
    )(p1, p2, x_s, xv48)


def _split_weights(params):
    p0, p1, p2 = params['layer0'], params['layer1'], params['layer2']
    eye3 = jnp.eye(3, dtype=jnp.float32)
    ws0 = p0['ws_w']
    w_ssrc = ws0[:SI]
    w_se = ws0[SI:SI + SE]
    w_sdst = ws0[SI + SE:SI + SE + SI]
    w_svn = _pad2(ws0[SI + SE + SI:], H0P, SO)
    wh0 = p0['wh']
    whs = _pad2(wh0[:VI], VI, H0P)
    whe = _pad2(wh0[VI:VI + VE], VE, H0P)
    whd = _pad2(wh0[VI + VE:], VI, H0P)
    wv0 = _pad2(p0['wv'], H0P, VO)
    wbig = jnp.concatenate([jnp.kron(eye3, whs), jnp.kron(eye3, whd),
                            jnp.zeros((32, 3 * H0P), jnp.float32)], axis=0)
    we = jnp.kron(eye3, whe)
    s0m = jnp.tile(jnp.eye(H0P, dtype=jnp.float32), (3, 1))
    s1m = jnp.tile(jnp.eye(VO, dtype=jnp.float32), (3, 1))
    wts = (
        w_se, p0['ws_b'][None, :], wbig, we, s0m, w_svn,
        jnp.kron(eye3, wv0),
        jnp.tile(p0['wsv_w'], (1, 3)),
        jnp.tile(p0['wsv_b'], (3,))[None, :],
        jnp.kron(eye3, p1['wh']),
        p1['ws_w'][:SO], p1['ws_w'][SO:], p1['ws_b'][None, :],
        jnp.kron(eye3, p1['wv']),
        jnp.tile(p1['wsv_w'], (1, 3)), jnp.tile(p1['wsv_b'], (3,))[None, :],
        s1m,
        jnp.kron(eye3, p2['wh']),
        p2['ws_w'][:SO], p2['ws_w'][SO:], p2['ws_b'][None, :],
        jnp.kron(eye3, p2['wv']),
        jnp.tile(p2['wsv_w'], (1, 3)), jnp.tile(p2['wsv_b'], (3,))[None, :],
    )
    return w_ssrc, w_sdst, wts


def kernel(x_s, x_v, edge_index, edge_attr_s, edge_attr_v, params):
    src, dst = edge_index[0], edge_index[1]
    w_ssrc, w_sdst, wts = _split_weights(params)

    xv48 = jnp.swapaxes(x_v, 1, 2).reshape(N, 48)
    ts_src, ts_dst = _node_tables(x_s, xv48, w_ssrc, w_sdst)
    eav = edge_attr_v.reshape(E, 3)

    gs, gv = _sc_gather(ts_src, ts_dst, src, dst, E)
    m1, m2 = _edge_gvp(gs, gv, edge_attr_s, eav, wts, E)
    p1, p2 = _sc_scatter(m1, m2, dst.reshape(SCHUNKS, SK))

    out_s, out_v48 = _combine(p1, p2, x_s, xv48)
    out_v = jnp.swapaxes(out_v48.reshape(N, 3, VI), 1, 2)
    return (out_s, out_v)


# speedup vs baseline: 1.1210x; 1.0297x over previous
import functools

import jax
import jax.numpy as jnp
from jax import lax
from jax.experimental import pallas as pl
from jax.experimental.pallas import tpu as pltpu
from jax.experimental.pallas import tpu_sc as plsc

N = 10000
E = 160000
SI, VI = 128, 16
SE, VE = 16, 1
SO, VO = 128, 16
H0 = 2 * VI + VE
H0P = 48
ROW = 192

EDGE_BLK = 4000
NODE_BLK = 1000


def _pad2(a, r, c):
    return jnp.pad(a, ((0, r - a.shape[0]), (0, c - a.shape[1])))


TBL = 176


def _node_kernel(xs_ref, xv_ref, wsrc_ref, wdst_ref, osrc_ref, odst_ref):
    xs = xs_ref[...]
    xv = xv_ref[...]
    osrc_ref[:, 0:128] = jnp.dot(xs, wsrc_ref[...], preferred_element_type=jnp.float32)
    osrc_ref[:, 128:176] = xv
    odst_ref[:, 0:128] = jnp.dot(xs, wdst_ref[...], preferred_element_type=jnp.float32)
    odst_ref[:, 128:176] = xv


def _node_tables(x_s, xv48, w_ssrc, w_sdst, interpret=False):
    grid = (N // NODE_BLK,)
    return pl.pallas_call(
        _node_kernel,
        grid=grid,
        in_specs=[
            pl.BlockSpec((NODE_BLK, SI), lambda i: (i, 0)),
            pl.BlockSpec((NODE_BLK, 48), lambda i: (i, 0)),
            pl.BlockSpec((SI, SO), lambda i: (0, 0)),
            pl.BlockSpec((SI, SO), lambda i: (0, 0)),
        ],
        out_specs=[
            pl.BlockSpec((NODE_BLK, TBL), lambda i: (i, 0)),
            pl.BlockSpec((NODE_BLK, TBL), lambda i: (i, 0)),
        ],
        out_shape=[
            jax.ShapeDtypeStruct((N, TBL), jnp.float32),
            jax.ShapeDtypeStruct((N, TBL), jnp.float32),
        ],
        interpret=interpret,
    )(x_s, xv48, w_ssrc, w_sdst)


GK = 128
NCHUNK = E // GK
_NC, _NS = 2, 16
_NW = _NC * _NS
_ITERS = (NCHUNK + _NW - 1) // _NW


_BASE_CH = NCHUNK // _NW
_EXTRA = NCHUNK - _BASE_CH * _NW


def _gather_body(nchunk, tsrc, tdst, src_hbm, dst_hbm, out_s, out_v,
                 idx_s0, idx_d0, idx_s1, idx_d1, bs0, bd0, bs1, bd1,
                 gsem0, gsem1, osem0, osem1):
    base = nchunk // _NW
    extra = nchunk % _NW
    wid = lax.axis_index("s") * _NC + lax.axis_index("c")
    nc = jnp.where(wid < extra, base + 1, base)
    start = base * wid + jnp.minimum(wid, extra)

    idx_s = (idx_s0, idx_s1)
    idx_d = (idx_d0, idx_d1)
    bs = (bs0, bs1)
    bd = (bd0, bd1)
    gsem = (gsem0, gsem1)
    osem = (osem0, osem1)

    def load_idx(c, p):
        off = (start + c) * GK
        pltpu.sync_copy(src_hbm.at[pl.ds(off, GK)], idx_s[p])
        pltpu.sync_copy(dst_hbm.at[pl.ds(off, GK)], idx_d[p])

    def start_gather(p):
        pltpu.async_copy(tsrc.at[idx_s[p]], bs[p], gsem[p])
        pltpu.async_copy(tdst.at[idx_d[p]], bd[p], gsem[p])

    def wait_gather(p):
        pltpu.make_async_copy(tsrc.at[idx_s[p]], bs[p], gsem[p]).wait()
        pltpu.make_async_copy(tdst.at[idx_d[p]], bd[p], gsem[p]).wait()

    def tec(p):
        b_s, b_d = bs[p], bd[p]

        zero16 = jnp.zeros((16,), jnp.float32)

        def row(k, _):
            for l in range(8):
                sl = pl.ds(16 * l, 16)
                b_s[k, sl] = b_s[k, sl] + b_d[k, sl]
            for l in range(3):
                s_sl = pl.ds(128 + 16 * l, 16)
                b_d[k, pl.ds(16 * l, 16)] = b_s[k, s_sl]
                b_d[k, pl.ds(48 + 16 * l, 16)] = b_d[k, s_sl]
            b_d[k, pl.ds(96, 16)] = zero16
            b_d[k, pl.ds(112, 16)] = zero16
            return 0

        lax.fori_loop(0, GK, row, 0)

    def start_out(c, p):
        off = (start + c) * GK
        pltpu.async_copy(bs[p].at[:, pl.ds(0, 128)],
                         out_s.at[pl.ds(off, GK)], osem[p])
        pltpu.async_copy(bd[p].at[:, pl.ds(0, 128)],
                         out_v.at[pl.ds(off, GK)], osem[p])

    def wait_out(p):
        pltpu.make_async_copy(bs[p].at[:, pl.ds(0, 128)],
                              out_s.at[pl.ds(0, GK)], osem[p]).wait()
        pltpu.make_async_copy(bd[p].at[:, pl.ds(0, 128)],
                              out_v.at[pl.ds(0, GK)], osem[p]).wait()

    load_idx(0, 0)
    start_gather(0)

    def half(i, p):
        @pl.when(i < nc)
        def _():
            @pl.when(i >= 1)
            def _():
                wait_out(1 - p)

            @pl.when(i + 1 < nc)
            def _():
                load_idx(i + 1, 1 - p)
                start_gather(1 - p)

            wait_gather(p)
            tec(p)
            start_out(i, p)

    def body2(i2, _):
        half(2 * i2, 0)
        half(2 * i2 + 1, 1)
        return 0

    maxnc = base + (1 if extra else 0)
    lax.fori_loop(0, (maxnc + 1) // 2, body2, 0)

    last = (nc - 1) % 2

    @pl.when(last == 0)
    def _():
        wait_out(0)

    @pl.when(last == 1)
    def _():
        wait_out(1)


def _sc_gather(tsrc, tdst, src, dst, e):
    f32 = jnp.float32
    return pl.kernel(
        functools.partial(_gather_body, e // GK),
        out_type=[
            jax.ShapeDtypeStruct((e, 128), f32),
            jax.ShapeDtypeStruct((e, 128), f32),
        ],
        mesh=plsc.VectorSubcoreMesh(core_axis_name="c", subcore_axis_name="s"),
        scratch_types=[
            pltpu.VMEM((GK,), jnp.int32),
            pltpu.VMEM((GK,), jnp.int32),
            pltpu.VMEM((GK,), jnp.int32),
            pltpu.VMEM((GK,), jnp.int32),
            pltpu.VMEM((GK, TBL), f32),
            pltpu.VMEM((GK, TBL), f32),
            pltpu.VMEM((GK, TBL), f32),
            pltpu.VMEM((GK, TBL), f32),
            pltpu.SemaphoreType.DMA,
            pltpu.SemaphoreType.DMA,
            pltpu.SemaphoreType.DMA,
            pltpu.SemaphoreType.DMA,
        ],
        compiler_params=pltpu.CompilerParams(use_tc_tiling_on_sc=False),
    )(tsrc, tdst, src, dst)


def _edge_kernel(gs_ref, gv_ref, eas_ref, eav_ref,
                 w_se_ref, b0_ref, wbig_ref, we_ref, s0m_ref, wsvn_ref,
                 wv0_ref, wsv0_ref, bsv0_ref,
                 wh1_ref, ws1_ref, wvn1_ref, b1_ref, wv1_ref, wsv1_ref, bsv1_ref,
                 s1m_ref,
                 wh2_ref, ws2_ref, wvn2_ref, b2_ref, wv2_ref, wsv2_ref, bsv2_ref,
                 out1_ref, out2_ref):
    f32 = jnp.float32

    def dot(a, b):
        return jnp.dot(a, b, preferred_element_type=f32)

    def sig(x):
        return 1.0 / (1.0 + jnp.exp(-x))

    gs = gs_ref[...]
    gv = gv_ref[...]
    eas = eas_ref[...]
    eav = eav_ref[...]

    vh = dot(gv, wbig_ref[...]) + dot(eav, we_ref[...])
    vn = jnp.sqrt(jnp.clip(dot(vh * vh, s0m_ref[...]), 1e-8, None))
    s0 = (gs + dot(eas, w_se_ref[...]) + dot(vn, wsvn_ref[...]) + b0_ref[...])
    gate0 = sig(dot(sig(s0), wsv0_ref[...]) + bsv0_ref[...])
    v0 = dot(vh, wv0_ref[...]) * gate0
    s0 = jax.nn.relu(s0)

    vh1 = dot(v0, wh1_ref[...])
    vn1 = jnp.sqrt(jnp.clip(dot(vh1 * vh1, s1m_ref[...]), 1e-8, None))
    s1 = dot(s0, ws1_ref[...]) + dot(vn1, wvn1_ref[...]) + b1_ref[...]
    gate1 = sig(dot(sig(s1), wsv1_ref[...]) + bsv1_ref[...])
    v1 = dot(vh1, wv1_ref[...]) * gate1
    s1 = jax.nn.relu(s1)

    vh2 = dot(v1, wh2_ref[...])
    vn2 = jnp.sqrt(jnp.clip(dot(vh2 * vh2, s1m_ref[...]), 1e-8, None))
    s2 = dot(s1, ws2_ref[...]) + dot(vn2, wvn2_ref[...]) + b2_ref[...]
    gate2 = sig(dot(s2, wsv2_ref[...]) + bsv2_ref[...])
    v2 = dot(vh2, wv2_ref[...]) * gate2

    out1_ref[...] = s2
    out2_ref[...] = jnp.concatenate(
        [v2, jnp.ones((s2.shape[0], 16), f32),
         jnp.zeros((s2.shape[0], 64), f32)], axis=1)


def _edge_gvp(gs, gv, eas, eav, wts, e=E, interpret=False):
    B = EDGE_BLK
    grid = (e // B,)
    full = lambda s: pl.BlockSpec(s, lambda i: (0, 0))
    in_specs = [
        pl.BlockSpec((B, SO), lambda i: (i, 0)),
        pl.BlockSpec((B, 128), lambda i: (i, 0)),
        pl.BlockSpec((B, SE), lambda i: (i, 0)),
        pl.BlockSpec((B, 3), lambda i: (i, 0)),
    ] + [full(w.shape) for w in wts]
    return pl.pallas_call(
        _edge_kernel,
        grid=grid,
        in_specs=in_specs,
        out_specs=[
            pl.BlockSpec((B, 128), lambda i: (i, 0)),
            pl.BlockSpec((B, 128), lambda i: (i, 0)),
        ],
        out_shape=[
            jax.ShapeDtypeStruct((e, 128), jnp.float32),
            jax.ShapeDtypeStruct((e, 128), jnp.float32),
        ],
        interpret=interpret,
    )(gs, gv, eas, eav, *wts)


SK = 128
SCHUNKS = E // SK
_SITERS = (SCHUNKS + _NS - 1) // _NS
NPT = N // _NS
NZC = 125


def _scatter_body(nhchunk, m1, m2, d2, o1, o2,
                  idx0, idx1, buf0, buf1, stage, acc, msem0, msem1):
    sbase = nhchunk // _NS
    sextra = nhchunk % _NS
    c = lax.axis_index("c")
    s = lax.axis_index("s")
    nc = jnp.where(s < sextra, sbase + 1, sbase)
    start = sbase * s + jnp.minimum(s, sextra)

    idx = (idx0, idx1)
    buf = (buf0, buf1)
    msem = (msem0, msem1)

    def zrow(k, _):
        for l in range(8):
            stage[k, pl.ds(16 * l, 16)] = jnp.zeros((16,), jnp.float32)
        return 0

    lax.fori_loop(0, NZC, zrow, 0)
    for j in range(NPT // NZC):
        pltpu.sync_copy(stage, acc.at[pl.ds(NPT * s + NZC * j, NZC)])
    plsc.subcore_barrier()

    def accumulate(m):
        def load(i, p):
            cid = start + i
            pltpu.sync_copy(d2.at[cid], idx[p])
            pltpu.async_copy(m.at[pl.ds(cid * SK, SK)], buf[p], msem[p])

        def waitm(p):
            pltpu.make_async_copy(m.at[pl.ds(0, SK)], buf[p], msem[p]).wait()

        load(0, 0)

        def half(i, p):
            @pl.when(i < nc)
            def _():
                @pl.when(i + 1 < nc)
                def _():
                    load(i + 1, 1 - p)

                waitm(p)
                pltpu.sync_copy(buf[p], acc.at[idx[p]], add=True)

        def body2(i2, _):
            half(2 * i2, 0)
            half(2 * i2 + 1, 1)
            return 0

        maxnc = sbase + (1 if sextra else 0)
        lax.fori_loop(0, (maxnc + 1) // 2, body2, 0)

    @pl.when(c == 0)
    def _():
        accumulate(m1)

    @pl.when(c == 1)
    def _():
        accumulate(m2)

    plsc.subcore_barrier()

    def writeout(o):
        for j in range(NPT // NZC):
            sl = pl.ds(NPT * s + NZC * j, NZC)
            pltpu.sync_copy(acc.at[sl], stage)
            pltpu.sync_copy(stage, o.at[sl])

    @pl.when(c == 0)
    def _():
        writeout(o1)

    @pl.when(c == 1)
    def _():
        writeout(o2)


def _sc_scatter(m1, m2, d2):
    f32 = jnp.float32
    return pl.kernel(
        functools.partial(_scatter_body, SCHUNKS),
        out_type=[
            jax.ShapeDtypeStruct((N, 128), f32),
            jax.ShapeDtypeStruct((N, 128), f32),
        ],
        mesh=plsc.VectorSubcoreMesh(core_axis_name="c", subcore_axis_name="s"),
        scratch_types=[
            pltpu.VMEM((SK,), jnp.int32),
            pltpu.VMEM((SK,), jnp.int32),
            pltpu.VMEM((SK, 128), f32),
            pltpu.VMEM((SK, 128), f32),
            pltpu.VMEM((NZC, 128), f32),
            pltpu.VMEM_SHARED((N, 128), f32),
            pltpu.SemaphoreType.DMA,
            pltpu.SemaphoreType.DMA,
        ],
        compiler_params=pltpu.CompilerParams(use_tc_tiling_on_sc=False),
    )(m1, m2, d2)


def _combine_kernel(p1_ref, p2_ref, xs_ref, xv_ref, os_ref, ov_ref):
    p1 = p1_ref[...]
    p2 = p2_ref[...]
    cnt = jnp.clip(p2[:, 48:49], 1.0, None)
    recip = 1.0 / cnt
    os_ref[...] = xs_ref[...] + p1 * recip
    ov_ref[...] = xv_ref[...] + p2[:, 0:48] * recip


def _combine(p1, p2, x_s, xv48, interpret=False):
    grid = (N // NODE_BLK,)
    return pl.pallas_call(
        _combine_kernel,
        grid=grid,
        in_specs=[
            pl.BlockSpec((NODE_BLK, 128), lambda i: (i, 0)),
            pl.BlockSpec((NODE_BLK, 128), lambda i: (i, 0)),
            pl.BlockSpec((NODE_BLK, SI), lambda i: (i, 0)),
            pl.BlockSpec((NODE_BLK, 48), lambda i: (i, 0)),
        ],
        out_specs=[
            pl.BlockSpec((NODE_BLK, SI), lambda i: (i, 0)),
            pl.BlockSpec((NODE_BLK, 48), lambda i: (i, 0)),
        ],
        out_shape=[
            jax.ShapeDtypeStruct((N, SI), jnp.float32),
            jax.ShapeDtypeStruct((N, 48), jnp.float32),
        ],
        interpret=interpret,
    )(p1, p2, x_s, xv48)


def _split_weights(params):
    p0, p1, p2 = params['layer0'], params['layer1'], params['layer2']
    eye3 = jnp.eye(3, dtype=jnp.float32)
    ws0 = p0['ws_w']
    w_ssrc = ws0[:SI]
    w_se = ws0[SI:SI + SE]
    w_sdst = ws0[SI + SE:SI + SE + SI]
    w_svn = _pad2(ws0[SI + SE + SI:], H0P, SO)
    wh0 = p0['wh']
    whs = _pad2(wh0[:VI], VI, H0P)
    whe = _pad2(wh0[VI:VI + VE], VE, H0P)
    whd = _pad2(wh0[VI + VE:], VI, H0P)
    wv0 = _pad2(p0['wv'], H0P, VO)
    wbig = jnp.concatenate([jnp.kron(eye3, whs), jnp.kron(eye3, whd),
                            jnp.zeros((32, 3 * H0P), jnp.float32)], axis=0)
    we = jnp.kron(eye3, whe)
    s0m = jnp.tile(jnp.eye(H0P, dtype=jnp.float32), (3, 1))
    s1m = jnp.tile(jnp.eye(VO, dtype=jnp.float32), (3, 1))
    wts = (
        w_se, p0['ws_b'][None, :], wbig, we, s0m, w_svn,
        jnp.kron(eye3, wv0),
        jnp.tile(p0['wsv_w'], (1, 3)),
        jnp.tile(p0['wsv_b'], (3,))[None, :],
        jnp.kron(eye3, p1['wh']),
        p1['ws_w'][:SO], p1['ws_w'][SO:], p1['ws_b'][None, :],
        jnp.kron(eye3, p1['wv']),
        jnp.tile(p1['wsv_w'], (1, 3)), jnp.tile(p1['wsv_b'], (3,))[None, :],
        s1m,
        jnp.kron(eye3, p2['wh']),
        p2['ws_w'][:SO], p2['ws_w'][SO:], p2['ws_b'][None, :],
        jnp.kron(eye3, p2['wv']),
        jnp.tile(p2['wsv_w'], (1, 3)), jnp.tile(p2['wsv_b'], (3,))[None, :],
    )
    return w_ssrc, w_sdst, wts


def kernel(x_s, x_v, edge_index, edge_attr_s, edge_attr_v, params):
    src, dst = edge_index[0], edge_index[1]
    w_ssrc, w_sdst, wts = _split_weights(params)

    xv48 = jnp.swapaxes(x_v, 1, 2).reshape(N, 48)
    ts_src, ts_dst = _node_tables(x_s, xv48, w_ssrc, w_sdst)
    eav = edge_attr_v.reshape(E, 3)

    gs, gv = _sc_gather(ts_src, ts_dst, src, dst, E)
    m1, m2 = _edge_gvp(gs, gv, edge_attr_s, eav, wts, E)
    p1, p2 = _sc_scatter(m1, m2, dst.reshape(SCHUNKS, SK))

    out_s, out_v48 = _combine(p1, p2, x_s, xv48)
    out_v = jnp.swapaxes(out_v48.reshape(N, 3, VI), 1, 2)
    return (out_s, out_v)
